# baseline (speedup 1.0000x reference)
import jax
import jax.numpy as jnp
from jax.experimental import pallas as pl
from jax.experimental.pallas import tpu as pltpu

S_BLK = 8


def _add_kernel(x_ref, emb_ref, out_ref):
    out_ref[...] = x_ref[...] + emb_ref[...][None, :, :]


def _part(x, embedding, s0, ns):
    batch, d_model = x.shape[1], x.shape[2]
    return pl.pallas_call(
        _add_kernel,
        grid=(ns // S_BLK,),
        in_specs=[
            pl.BlockSpec((S_BLK, batch, d_model), lambda i: (s0 // S_BLK + i, 0, 0)),
            pl.BlockSpec((batch, d_model), lambda i: (0, 0)),
        ],
        out_specs=pl.BlockSpec((S_BLK, batch, d_model), lambda i: (i, 0, 0)),
        out_shape=jax.ShapeDtypeStruct((ns, batch, d_model), x.dtype),
        compiler_params=pltpu.CompilerParams(
            dimension_semantics=("arbitrary",),
        ),
    )(x, embedding)


def kernel(x, embedding):
    head = _part(x, embedding, 0, 448)
    tail = _part(x, embedding, 448, 64)
    return jnp.concatenate([head, tail], axis=0)


# speedup vs baseline: 2.0188x; 2.0188x over previous
import jax
import jax.numpy as jnp
from jax.experimental import pallas as pl
from jax.experimental.pallas import tpu as pltpu

S_BLK = 4


def _add_kernel(x_ref, emb_ref, out_ref):
    out_ref[...] = x_ref[...] + emb_ref[...][None, :, :]


def kernel(x, embedding):
    seq_len, batch, d_model = x.shape
    grid = (seq_len // S_BLK,)
    return pl.pallas_call(
        _add_kernel,
        grid=grid,
        in_specs=[
            pl.BlockSpec((S_BLK, batch, d_model), lambda i: (i, 0, 0)),
            pl.BlockSpec((batch, d_model), lambda i: (0, 0)),
        ],
        out_specs=pl.BlockSpec((S_BLK, batch, d_model), lambda i: (i, 0, 0)),
        out_shape=jax.ShapeDtypeStruct((seq_len, batch, d_model), x.dtype),
        compiler_params=pltpu.CompilerParams(
            dimension_semantics=("arbitrary",),
        ),
    )(x, embedding)
